# Initial kernel scaffold; baseline (speedup 1.0000x reference)
import jax
import jax.numpy as jnp
from jax.experimental import pallas as pl


def kernel(xyz1, xyz2, feat1, feat2, W0, b0, g0, be0, W1, b1, g1, be1):
    raise NotImplementedError("write your pallas kernel here")


# speedup vs baseline: 40.2890x; 40.2890x over previous
import functools

import jax
import jax.numpy as jnp
from jax.experimental import pallas as pl


def _k1_body(xyz1_ref, xyz2t_ref, feat1_ref, feat2_ref, w0t_ref, b0_ref,
             out_ref, stats_ref):
    b = pl.program_id(0)
    i = pl.program_id(1)

    x1 = xyz1_ref[0]
    x2t = xyz2t_ref[0]
    n2 = x2t.shape[1]

    dot = jnp.dot(x1, x2t, preferred_element_type=jnp.float32)
    x1s = jnp.sum(x1 * x1, axis=1, keepdims=True)
    x2s = jnp.sum(x2t * x2t, axis=0, keepdims=True)
    sq = jnp.maximum(x1s + x2s - 2.0 * dot, 1e-12)

    cols = jax.lax.broadcasted_iota(jnp.int32, sq.shape, 1)
    big = jnp.float32(3.0e38)

    m1 = jnp.min(sq, axis=1, keepdims=True)
    i1 = jnp.min(jnp.where(sq == m1, cols, n2), axis=1, keepdims=True)
    sqm = jnp.where(cols == i1, big, sq)
    m2 = jnp.min(sqm, axis=1, keepdims=True)
    i2 = jnp.min(jnp.where(sqm == m2, cols, n2), axis=1, keepdims=True)
    sqm = jnp.where(cols == i2, big, sqm)
    m3 = jnp.min(sqm, axis=1, keepdims=True)
    i3 = jnp.min(jnp.where(sqm == m3, cols, n2), axis=1, keepdims=True)

    r1 = 1.0 / (jnp.sqrt(m1) + 1e-8)
    r2 = 1.0 / (jnp.sqrt(m2) + 1e-8)
    r3 = 1.0 / (jnp.sqrt(m3) + 1e-8)
    s = r1 + r2 + r3
    w1 = r1 / s
    w2 = r2 / s
    w3 = r3 / s

    ws = (jnp.where(cols == i1, w1, 0.0)
          + jnp.where(cols == i2, w2, 0.0)
          + jnp.where(cols == i3, w3, 0.0))

    interp = jnp.dot(ws, feat2_ref[0], preferred_element_type=jnp.float32)
    c1 = feat1_ref.shape[2]
    x = (jnp.dot(feat1_ref[0], w0t_ref[:c1], preferred_element_type=jnp.float32)
         + jnp.dot(interp, w0t_ref[c1:], preferred_element_type=jnp.float32)
         + b0_ref[...])
    out_ref[0] = x

    ps = jnp.sum(x, axis=0, keepdims=True)
    pss = jnp.sum(x * x, axis=0, keepdims=True)
    upd = jnp.concatenate([ps, pss, jnp.zeros((6, x.shape[1]), jnp.float32)],
                          axis=0)

    @pl.when(jnp.logical_and(b == 0, i == 0))
    def _():
        stats_ref[...] = jnp.zeros_like(stats_ref)

    stats_ref[...] += upd


def _k2_body(x_ref, sc_ref, sh_ref, w1t_ref, b1_ref, out_ref, stats_ref):
    x = jnp.maximum(x_ref[...] * sc_ref[...] + sh_ref[...], 0.0)
    y = jnp.dot(x, w1t_ref[...], preferred_element_type=jnp.float32) + b1_ref[...]
    out_ref[...] = y

    ps = jnp.sum(y, axis=0, keepdims=True)
    pss = jnp.sum(y * y, axis=0, keepdims=True)
    upd = jnp.concatenate([ps, pss, jnp.zeros((6, y.shape[1]), jnp.float32)],
                          axis=0)

    @pl.when(pl.program_id(0) == 0)
    def _():
        stats_ref[...] = jnp.zeros_like(stats_ref)

    stats_ref[...] += upd


def _k3_body(x_ref, sc_ref, sh_ref, out_ref):
    out_ref[...] = jnp.maximum(x_ref[...] * sc_ref[...] + sh_ref[...], 0.0)


@jax.jit
def kernel(xyz1, xyz2, feat1, feat2, W0, b0, g0, be0, W1, b1, g1, be1):
    B, N1, _ = xyz1.shape
    N2 = xyz2.shape[1]
    C1 = feat1.shape[2]
    C2 = feat2.shape[2]
    H0 = W0.shape[0]
    H1 = W1.shape[0]
    M = B * N1

    T1 = min(256, N1)
    xyz2t = jnp.swapaxes(xyz2, 1, 2)
    w0t = W0.T
    w1t = W1.T

    x1_pre, stats0 = pl.pallas_call(
        _k1_body,
        grid=(B, N1 // T1),
        in_specs=[
            pl.BlockSpec((1, T1, 3), lambda b, i: (b, i, 0)),
            pl.BlockSpec((1, 3, N2), lambda b, i: (b, 0, 0)),
            pl.BlockSpec((1, T1, C1), lambda b, i: (b, i, 0)),
            pl.BlockSpec((1, N2, C2), lambda b, i: (b, 0, 0)),
            pl.BlockSpec((C1 + C2, H0), lambda b, i: (0, 0)),
            pl.BlockSpec((1, H0), lambda b, i: (0, 0)),
        ],
        out_specs=[
            pl.BlockSpec((1, T1, H0), lambda b, i: (b, i, 0)),
            pl.BlockSpec((8, H0), lambda b, i: (0, 0)),
        ],
        out_shape=[
            jax.ShapeDtypeStruct((B, N1, H0), jnp.float32),
            jax.ShapeDtypeStruct((8, H0), jnp.float32),
        ],
    )(xyz1, xyz2t, feat1, feat2, w0t, b0.reshape(1, H0))

    mu0 = stats0[0] / M
    var0 = stats0[1] / M - mu0 * mu0
    sc0 = (g0 / jnp.sqrt(var0 + 1e-5)).reshape(1, H0)
    sh0 = (be0 - mu0 * g0 / jnp.sqrt(var0 + 1e-5)).reshape(1, H0)

    T2 = min(2048, M)
    x1_flat = x1_pre.reshape(M, H0)
    x2_pre, stats1 = pl.pallas_call(
        _k2_body,
        grid=(M // T2,),
        in_specs=[
            pl.BlockSpec((T2, H0), lambda i: (i, 0)),
            pl.BlockSpec((1, H0), lambda i: (0, 0)),
            pl.BlockSpec((1, H0), lambda i: (0, 0)),
            pl.BlockSpec((H0, H1), lambda i: (0, 0)),
            pl.BlockSpec((1, H1), lambda i: (0, 0)),
        ],
        out_specs=[
            pl.BlockSpec((T2, H1), lambda i: (i, 0)),
            pl.BlockSpec((8, H1), lambda i: (0, 0)),
        ],
        out_shape=[
            jax.ShapeDtypeStruct((M, H1), jnp.float32),
            jax.ShapeDtypeStruct((8, H1), jnp.float32),
        ],
    )(x1_flat, sc0, sh0, w1t, b1.reshape(1, H1))

    mu1 = stats1[0] / M
    var1 = stats1[1] / M - mu1 * mu1
    sc1 = (g1 / jnp.sqrt(var1 + 1e-5)).reshape(1, H1)
    sh1 = (be1 - mu1 * g1 / jnp.sqrt(var1 + 1e-5)).reshape(1, H1)

    out = pl.pallas_call(
        _k3_body,
        grid=(M // T2,),
        in_specs=[
            pl.BlockSpec((T2, H1), lambda i: (i, 0)),
            pl.BlockSpec((1, H1), lambda i: (0, 0)),
            pl.BlockSpec((1, H1), lambda i: (0, 0)),
        ],
        out_specs=pl.BlockSpec((T2, H1), lambda i: (i, 0)),
        out_shape=jax.ShapeDtypeStruct((M, H1), jnp.float32),
    )(x2_pre, sc1, sh1)

    return out.reshape(B, N1, H1)


# speedup vs baseline: 58.7657x; 1.4586x over previous
import functools

import jax
import jax.numpy as jnp
from jax.experimental import pallas as pl


def _k1_body(xyz1_ref, xyz2t_ref, feat1_ref, feat2_ref, w0t_ref, b0_ref,
             out_ref, stats_ref):
    b = pl.program_id(0)
    i = pl.program_id(1)

    x1 = xyz1_ref[0]
    x2t = xyz2t_ref[0]
    t1 = x1.shape[0]
    n2 = x2t.shape[1]

    x1s = jnp.sum(x1 * x1, axis=1, keepdims=True)
    x2s = jnp.sum(x2t * x2t, axis=0, keepdims=True)
    a = jnp.concatenate([-2.0 * x1, jnp.ones((t1, 1), jnp.float32), x1s],
                        axis=1)
    bm = jnp.concatenate([x2t, x2s, jnp.ones((1, n2), jnp.float32)],
                         axis=0)
    sq = jnp.maximum(jnp.dot(a, bm, preferred_element_type=jnp.float32),
                     1e-12)

    big = jnp.float32(3.0e38)

    m1 = jnp.min(sq, axis=1, keepdims=True)
    sqm = jnp.where(sq > m1, sq, big)
    m2 = jnp.min(sqm, axis=1, keepdims=True)
    sqm = jnp.where(sqm > m2, sqm, big)
    m3 = jnp.min(sqm, axis=1, keepdims=True)

    r1 = 1.0 / (jnp.sqrt(m1) + 1e-8)
    r2 = 1.0 / (jnp.sqrt(m2) + 1e-8)
    r3 = 1.0 / (jnp.sqrt(m3) + 1e-8)
    s = r1 + r2 + r3
    w1 = r1 / s
    w2 = r2 / s
    w3 = r3 / s

    ws = jnp.where(sq == m1, w1,
                   jnp.where(sq == m2, w2,
                             jnp.where(sq == m3, w3, 0.0)))

    interp = jnp.dot(ws, feat2_ref[0], preferred_element_type=jnp.float32)
    c1 = feat1_ref.shape[2]
    x = (jnp.dot(feat1_ref[0], w0t_ref[:c1], preferred_element_type=jnp.float32)
         + jnp.dot(interp, w0t_ref[c1:], preferred_element_type=jnp.float32)
         + b0_ref[...])
    out_ref[0] = x

    ps = jnp.sum(x, axis=0, keepdims=True)
    pss = jnp.sum(x * x, axis=0, keepdims=True)
    upd = jnp.concatenate([ps, pss, jnp.zeros((6, x.shape[1]), jnp.float32)],
                          axis=0)

    @pl.when(jnp.logical_and(b == 0, i == 0))
    def _():
        stats_ref[...] = jnp.zeros_like(stats_ref)

    stats_ref[...] += upd


def _k2_body(x_ref, sc_ref, sh_ref, w1t_ref, b1_ref, out_ref, stats_ref):
    x = jnp.maximum(x_ref[...] * sc_ref[...] + sh_ref[...], 0.0)
    y = jnp.dot(x, w1t_ref[...], preferred_element_type=jnp.float32) + b1_ref[...]
    out_ref[...] = y

    ps = jnp.sum(y, axis=0, keepdims=True)
    pss = jnp.sum(y * y, axis=0, keepdims=True)
    upd = jnp.concatenate([ps, pss, jnp.zeros((6, y.shape[1]), jnp.float32)],
                          axis=0)

    @pl.when(pl.program_id(0) == 0)
    def _():
        stats_ref[...] = jnp.zeros_like(stats_ref)

    stats_ref[...] += upd


def _k3_body(x_ref, sc_ref, sh_ref, out_ref):
    out_ref[...] = jnp.maximum(x_ref[...] * sc_ref[...] + sh_ref[...], 0.0)


@jax.jit
def kernel(xyz1, xyz2, feat1, feat2, W0, b0, g0, be0, W1, b1, g1, be1):
    B, N1, _ = xyz1.shape
    N2 = xyz2.shape[1]
    C1 = feat1.shape[2]
    C2 = feat2.shape[2]
    H0 = W0.shape[0]
    H1 = W1.shape[0]
    M = B * N1

    T1 = min(256, N1)
    xyz2t = jnp.swapaxes(xyz2, 1, 2)
    w0t = W0.T
    w1t = W1.T

    x1_pre, stats0 = pl.pallas_call(
        _k1_body,
        grid=(B, N1 // T1),
        in_specs=[
            pl.BlockSpec((1, T1, 3), lambda b, i: (b, i, 0)),
            pl.BlockSpec((1, 3, N2), lambda b, i: (b, 0, 0)),
            pl.BlockSpec((1, T1, C1), lambda b, i: (b, i, 0)),
            pl.BlockSpec((1, N2, C2), lambda b, i: (b, 0, 0)),
            pl.BlockSpec((C1 + C2, H0), lambda b, i: (0, 0)),
            pl.BlockSpec((1, H0), lambda b, i: (0, 0)),
        ],
        out_specs=[
            pl.BlockSpec((1, T1, H0), lambda b, i: (b, i, 0)),
            pl.BlockSpec((8, H0), lambda b, i: (0, 0)),
        ],
        out_shape=[
            jax.ShapeDtypeStruct((B, N1, H0), jnp.float32),
            jax.ShapeDtypeStruct((8, H0), jnp.float32),
        ],
    )(xyz1, xyz2t, feat1, feat2, w0t, b0.reshape(1, H0))

    mu0 = stats0[0] / M
    var0 = stats0[1] / M - mu0 * mu0
    sc0 = (g0 / jnp.sqrt(var0 + 1e-5)).reshape(1, H0)
    sh0 = (be0 - mu0 * g0 / jnp.sqrt(var0 + 1e-5)).reshape(1, H0)

    T2 = min(2048, M)
    x1_flat = x1_pre.reshape(M, H0)
    x2_pre, stats1 = pl.pallas_call(
        _k2_body,
        grid=(M // T2,),
        in_specs=[
            pl.BlockSpec((T2, H0), lambda i: (i, 0)),
            pl.BlockSpec((1, H0), lambda i: (0, 0)),
            pl.BlockSpec((1, H0), lambda i: (0, 0)),
            pl.BlockSpec((H0, H1), lambda i: (0, 0)),
            pl.BlockSpec((1, H1), lambda i: (0, 0)),
        ],
        out_specs=[
            pl.BlockSpec((T2, H1), lambda i: (i, 0)),
            pl.BlockSpec((8, H1), lambda i: (0, 0)),
        ],
        out_shape=[
            jax.ShapeDtypeStruct((M, H1), jnp.float32),
            jax.ShapeDtypeStruct((8, H1), jnp.float32),
        ],
    )(x1_flat, sc0, sh0, w1t, b1.reshape(1, H1))

    mu1 = stats1[0] / M
    var1 = stats1[1] / M - mu1 * mu1
    sc1 = (g1 / jnp.sqrt(var1 + 1e-5)).reshape(1, H1)
    sh1 = (be1 - mu1 * g1 / jnp.sqrt(var1 + 1e-5)).reshape(1, H1)

    out = pl.pallas_call(
        _k3_body,
        grid=(M // T2,),
        in_specs=[
            pl.BlockSpec((T2, H1), lambda i: (i, 0)),
            pl.BlockSpec((1, H1), lambda i: (0, 0)),
            pl.BlockSpec((1, H1), lambda i: (0, 0)),
        ],
        out_specs=pl.BlockSpec((T2, H1), lambda i: (i, 0)),
        out_shape=jax.ShapeDtypeStruct((M, H1), jnp.float32),
    )(x2_pre, sc1, sh1)

    return out.reshape(B, N1, H1)
